# baseline (speedup 1.0000x reference)
import functools

import jax
import jax.numpy as jnp
import numpy as np
from jax import lax
from jax.experimental import pallas as pl
from jax.experimental.pallas import tpu as pltpu
from jax.experimental.pallas import tpu_sc as plsc

_N = 1024
_K = 64
_B = 8
_NS = 32
_EPS_TERM = float(_B * _K * 1e-31)
_BIAS = float(_N * np.log(1.0 / _B))

_NBLK = 128

_NC = 2
_NSUB = 16
_NW = _NC * _NSUB
_NPT = _N // _NW

_LUT = np.log(np.arange(16, dtype=np.float64) + _EPS_TERM).astype(np.float32)


def _tc_index_body(sample_ref, logits_ref, comp_ref, ft_ref, g_ref):
    i = pl.program_id(0)
    lg = logits_ref[...]
    kio = lax.broadcasted_iota(jnp.int32, (_NBLK, _B, _K), 2)
    mx = jnp.max(lg, axis=-1, keepdims=True)
    m = jnp.min(jnp.where(lg == mx, kio, _K), axis=-1)
    cp = comp_ref[...]
    c = jnp.sum(cp * kio.astype(jnp.float32), axis=-1).astype(jnp.int32)
    t = (c - m + _K) & (_K - 1)
    n_row = (lax.broadcasted_iota(jnp.int32, (_B, _NBLK), 1) + i * _NBLK) * _K
    ft_ref[...] = jnp.transpose(t) + n_row
    smp = sample_ref[...]
    kio_s = lax.broadcasted_iota(jnp.int32, (_NS, _NBLK, _K), 2)
    s = jnp.sum(smp * kio_s.astype(jnp.float32),
                axis=-1).astype(jnp.int32)
    n_col = (lax.broadcasted_iota(jnp.int32, (_NBLK, _NS), 0) + i * _NBLK) * _K
    g_ref[...] = jnp.transpose(s) + n_col


def _tc_index_stage(sample, logits, component_probs):
    return pl.pallas_call(
        _tc_index_body,
        grid=(_N // _NBLK,),
        in_specs=[
            pl.BlockSpec((_NS, _NBLK, _K), lambda i: (0, i, 0)),
            pl.BlockSpec((_NBLK, _B, _K), lambda i: (i, 0, 0)),
            pl.BlockSpec((_NBLK, _B, _K), lambda i: (i, 0, 0)),
        ],
        out_specs=[
            pl.BlockSpec((_B, _NBLK), lambda i: (0, i)),
            pl.BlockSpec((_NBLK, _NS), lambda i: (i, 0)),
        ],
        out_shape=[
            jax.ShapeDtypeStruct((_B, _N), jnp.int32),
            jax.ShapeDtypeStruct((_N, _NS), jnp.int32),
        ],
    )(sample, logits, component_probs)


def _sc_body(ft_hbm, g_hbm, lut_hbm, out_hbm, fv, gv, lut_v, tbl, acc_v):
    cid = lax.axis_index("c")
    sid = lax.axis_index("s")
    wid = sid * _NC + cid
    n0 = wid * _NPT
    base = n0 * _K

    pltpu.sync_copy(lut_hbm, lut_v)
    pltpu.sync_copy(g_hbm.at[pl.ds(n0 * _NS, _NPT * _NS)], gv)
    for b in range(_B):
        pltpu.sync_copy(ft_hbm.at[pl.ds(b * _N + n0, _NPT)],
                        fv.at[pl.ds(b * _NPT, _NPT)])

    zeros = jnp.zeros((16,), jnp.float32)
    for i in range(_NPT * _K // 16):
        tbl[pl.ds(i * 16, 16)] = zeros

    ones = jnp.ones((16,), jnp.float32)
    for b in range(_B):
        for h in range(_NPT // 16):
            idx = fv[pl.ds(b * _NPT + h * 16, 16)] - base
            plsc.addupdate_scatter(tbl, [idx], ones)

    acc0 = jnp.zeros((16,), jnp.float32)
    acc1 = jnp.zeros((16,), jnp.float32)
    for n in range(_NPT):
        g0 = gv[pl.ds(n * _NS, 16)] - base
        g1 = gv[pl.ds(n * _NS + 16, 16)] - base
        c0 = plsc.load_gather(tbl, [g0]).astype(jnp.int32)
        c1 = plsc.load_gather(tbl, [g1]).astype(jnp.int32)
        acc0 = acc0 + plsc.load_gather(lut_v, [c0])
        acc1 = acc1 + plsc.load_gather(lut_v, [c1])
    acc_v[pl.ds(0, 16)] = acc0
    acc_v[pl.ds(16, 16)] = acc1
    pltpu.sync_copy(acc_v, out_hbm.at[wid])


def _sc_stage(ft_flat, g_flat, lut):
    fn = pl.kernel(
        _sc_body,
        out_type=jax.ShapeDtypeStruct((_NW, _NS), jnp.float32),
        mesh=plsc.VectorSubcoreMesh(core_axis_name="c", subcore_axis_name="s",
                                    num_cores=_NC, num_subcores=_NSUB),
        scratch_types=[
            pltpu.VMEM((_B * _NPT,), jnp.int32),
            pltpu.VMEM((_NPT * _NS,), jnp.int32),
            pltpu.VMEM((16,), jnp.float32),
            pltpu.VMEM((_NPT * _K,), jnp.float32),
            pltpu.VMEM((_NS,), jnp.float32),
        ],
        compiler_params=pltpu.CompilerParams(needs_layout_passes=False),
    )
    return fn(ft_flat, g_flat, lut)


def _tc_reduce_body(part_ref, out_ref):
    out_ref[...] = jnp.sum(part_ref[...], axis=0, keepdims=True) + _BIAS


def _tc_reduce_stage(partial):
    return pl.pallas_call(
        _tc_reduce_body,
        out_shape=jax.ShapeDtypeStruct((1, _NS), jnp.float32),
    )(partial)


@jax.jit
def kernel(sample, logits, component_probs):
    ft2d, g2d = _tc_index_stage(sample, logits, component_probs)
    lut = jnp.asarray(_LUT)
    partial = _sc_stage(ft2d.reshape(_B * _N), g2d.reshape(_N * _NS), lut)
    out = _tc_reduce_stage(partial)
    return out.reshape(_NS)


# speedup vs baseline: 1.0562x; 1.0562x over previous
import functools

import jax
import jax.numpy as jnp
import numpy as np
from jax import lax
from jax.experimental import pallas as pl
from jax.experimental.pallas import tpu as pltpu
from jax.experimental.pallas import tpu_sc as plsc

_N = 1024
_K = 64
_B = 8
_NS = 32
_EPS_TERM = float(_B * _K * 1e-31)
_BIAS = float(_N * np.log(1.0 / _B))

_NBLK = 128

_NC = 2
_NSUB = 16
_NW = _NC * _NSUB
_NPT = _N // _NW

_LUT = np.log(np.arange(16, dtype=np.float64) + _EPS_TERM).astype(np.float32)


def _tc_index_body(sample_ref, logits_ref, comp_ref, ft_ref, g_ref):
    lg = logits_ref[...]
    kio = lax.broadcasted_iota(jnp.int32, (_NBLK, _B, _K), 2)
    mx = jnp.max(lg, axis=-1, keepdims=True)
    m = jnp.min(jnp.where(lg == mx, kio, _K), axis=-1)
    cp = comp_ref[...]
    c = jnp.sum(cp * kio.astype(jnp.float32), axis=-1).astype(jnp.int32)
    t = (c - m + _K) & (_K - 1)
    nl = (lax.broadcasted_iota(jnp.int32, (_NBLK, _B), 0) % _NPT) * _K
    tt = t + nl
    ft_ref[...] = jnp.swapaxes(
        tt.reshape(_NBLK // _NPT, _NPT, _B), 1, 2
    ).reshape(_NBLK // _NPT, 1, _NPT * _B)
    smp = sample_ref[...]
    kio_s = lax.broadcasted_iota(jnp.int32, (_NS, _NBLK, _K), 2)
    s = jnp.sum(smp * kio_s.astype(jnp.float32),
                axis=-1).astype(jnp.int32)
    nc = (lax.broadcasted_iota(jnp.int32, (_NBLK, _NS), 0) % _NPT) * _K
    g_ref[...] = jnp.transpose(s) + nc


def _tc_index_stage(sample, logits, component_probs):
    return pl.pallas_call(
        _tc_index_body,
        grid=(_N // _NBLK,),
        in_specs=[
            pl.BlockSpec((_NS, _NBLK, _K), lambda i: (0, i, 0)),
            pl.BlockSpec((_NBLK, _B, _K), lambda i: (i, 0, 0)),
            pl.BlockSpec((_NBLK, _B, _K), lambda i: (i, 0, 0)),
        ],
        out_specs=[
            pl.BlockSpec((_NBLK // _NPT, 1, _NPT * _B), lambda i: (i, 0, 0)),
            pl.BlockSpec((_NBLK, _NS), lambda i: (i, 0)),
        ],
        out_shape=[
            jax.ShapeDtypeStruct((_NW, 1, _NPT * _B), jnp.int32),
            jax.ShapeDtypeStruct((_N, _NS), jnp.int32),
        ],
    )(sample, logits, component_probs)


def _sc_body(ft_hbm, g_hbm, lut_hbm, out_hbm,
             fv, gv, lut_v, tbl, acc_v, sem_f, sem_g, sem_l):
    cid = lax.axis_index("c")
    sid = lax.axis_index("s")
    wid = cid * _NSUB + sid

    cp_f = pltpu.async_copy(ft_hbm.at[pl.ds(wid * _NPT * _B, _NPT * _B)],
                            fv, sem_f)
    cp_g = pltpu.async_copy(g_hbm.at[pl.ds(wid * _NPT * _NS, _NPT * _NS)],
                            gv, sem_g)
    cp_l = pltpu.async_copy(lut_hbm, lut_v, sem_l)

    zeros = jnp.zeros((16,), jnp.float32)
    for i in range(_NPT * _K // 16):
        tbl[pl.ds(i * 16, 16)] = zeros

    cp_f.wait()
    ones = jnp.ones((16,), jnp.float32)
    for b in range(_B):
        for h in range(_NPT // 16):
            idx = fv[pl.ds(b * _NPT + h * 16, 16)]
            plsc.addupdate_scatter(tbl, [idx], ones)

    cp_l.wait()
    cp_g.wait()
    acc0 = jnp.zeros((16,), jnp.float32)
    acc1 = jnp.zeros((16,), jnp.float32)
    for n in range(_NPT):
        g0 = gv[pl.ds(n * _NS, 16)]
        g1 = gv[pl.ds(n * _NS + 16, 16)]
        c0 = plsc.load_gather(tbl, [g0]).astype(jnp.int32)
        c1 = plsc.load_gather(tbl, [g1]).astype(jnp.int32)
        acc0 = acc0 + plsc.load_gather(lut_v, [c0])
        acc1 = acc1 + plsc.load_gather(lut_v, [c1])
    acc_v[pl.ds(0, 16)] = acc0
    acc_v[pl.ds(16, 16)] = acc1
    pltpu.sync_copy(acc_v, out_hbm.at[wid])


def _sc_stage(ft2d, g_flat, lut):
    fn = pl.kernel(
        _sc_body,
        out_type=jax.ShapeDtypeStruct((_NW, _NS), jnp.float32),
        mesh=plsc.VectorSubcoreMesh(core_axis_name="c", subcore_axis_name="s",
                                    num_cores=_NC, num_subcores=_NSUB),
        scratch_types=[
            pltpu.VMEM((_B * _NPT,), jnp.int32),
            pltpu.VMEM((_NPT * _NS,), jnp.int32),
            pltpu.VMEM((16,), jnp.float32),
            pltpu.VMEM((_NPT * _K,), jnp.float32),
            pltpu.VMEM((_NS,), jnp.float32),
            pltpu.SemaphoreType.DMA,
            pltpu.SemaphoreType.DMA,
            pltpu.SemaphoreType.DMA,
        ],
        compiler_params=pltpu.CompilerParams(needs_layout_passes=False),
    )
    return fn(ft2d, g_flat, lut)


@jax.jit
def kernel(sample, logits, component_probs):
    ft2d, g2d = _tc_index_stage(sample, logits, component_probs)
    lut = jnp.asarray(_LUT)
    partial = _sc_stage(ft2d.reshape(_NW * _NPT * _B),
                        g2d.reshape(_N * _NS), lut)
    return jnp.sum(partial, axis=0) + _BIAS


# speedup vs baseline: 1.7684x; 1.6744x over previous
import functools

import jax
import jax.numpy as jnp
import numpy as np
from jax import lax
from jax.experimental import pallas as pl

_N = 1024
_K = 64
_B = 8
_NS = 32
_EPS_TERM = float(_B * _K * 1e-31)
_BIAS = float(_N * np.log(1.0 / _B))

_NBLK = 128


def _tc_body(sample_ref, logits_ref, comp_ref, out_ref):
    i = pl.program_id(0)
    lg = logits_ref[...]
    kio = lax.broadcasted_iota(jnp.int32, (_NBLK, _B, _K), 2)
    mx = jnp.max(lg, axis=-1, keepdims=True)
    m = jnp.min(jnp.where(lg == mx, kio, _K), axis=-1)
    cp = comp_ref[...]
    c = jnp.sum(cp * kio.astype(jnp.float32), axis=-1).astype(jnp.int32)
    t = (c - m + _K) & (_K - 1)
    T = jnp.sum((t[:, :, None] == kio).astype(jnp.float32), axis=1)
    smp = sample_ref[...]
    cnt = jnp.sum(smp * T[None, :, :], axis=-1)
    part = jnp.sum(jnp.log(cnt + _EPS_TERM), axis=1)

    @pl.when(i == 0)
    def _init():
        out_ref[...] = jnp.full((1, _NS), _BIAS, jnp.float32)

    out_ref[...] += part[None, :]


@jax.jit
def kernel(sample, logits, component_probs):
    grid = _N // _NBLK
    out = pl.pallas_call(
        _tc_body,
        grid=(grid,),
        in_specs=[
            pl.BlockSpec((_NS, _NBLK, _K), lambda i: (0, i, 0)),
            pl.BlockSpec((_NBLK, _B, _K), lambda i: (i, 0, 0)),
            pl.BlockSpec((_NBLK, _B, _K), lambda i: (i, 0, 0)),
        ],
        out_specs=pl.BlockSpec((1, _NS), lambda i: (0, 0)),
        out_shape=jax.ShapeDtypeStruct((1, _NS), jnp.float32),
    )(sample, logits, component_probs)
    return out.reshape(_NS)
